# baseline (speedup 1.0000x reference)
import jax
import jax.numpy as jnp
from jax.experimental import pallas as pl

N = 10000
H = 512
HEADS = 4
HD = H // HEADS


def _bn(x, g, b, eps=1e-5):
    mu = x.mean(0, keepdims=True)
    var = x.var(0, keepdims=True)
    return g * (x - mu) / jnp.sqrt(var + eps) + b


def _final_mm_kernel(c_ref, w_ref, b_ref, o_ref):
    o_ref[...] = c_ref[...] @ w_ref[...] + b_ref[...]


def kernel(x, edge_index, params):
    p = params
    h = jax.nn.relu(_bn(x @ p['W_in'] + p['b_in'], p['bn_in_g'], p['bn_in_b']))
    src, dst = edge_index[0], edge_index[1]
    loop = jnp.arange(N, dtype=src.dtype)
    src_sl = jnp.concatenate([src, loop])
    dst_sl = jnp.concatenate([dst, loop])
    hg = (h @ p['W_gat']).reshape(N, HEADS, HD)
    a_s = (hg * p['att_src'][None]).sum(-1)
    a_d = (hg * p['att_dst'][None]).sum(-1)
    e = jax.nn.leaky_relu(a_s[src_sl] + a_d[dst_sl], 0.2)
    m = jax.ops.segment_max(e, dst_sl, num_segments=N)
    m = jnp.where(jnp.isfinite(m), m, 0.0)
    ex = jnp.exp(e - m[dst_sl])
    denom = jax.ops.segment_sum(ex, dst_sl, num_segments=N)
    alpha = ex / (denom[dst_sl] + 1e-16)
    gat_out = jax.ops.segment_sum(hg[src_sl] * alpha[:, :, None], dst_sl, num_segments=N).reshape(N, H) + p['b_gat']
    hc = h @ p['W_gcn']
    deg = jax.ops.segment_sum(jnp.ones(src_sl.shape[0], jnp.float32), dst_sl, num_segments=N)
    dinv = jax.lax.rsqrt(jnp.clip(deg, 1.0))
    norm = dinv[src_sl] * dinv[dst_sl]
    gcn_out = jax.ops.segment_sum(hc[src_sl] * norm[:, None], dst_sl, num_segments=N) + p['b_gcn']
    ssum = jax.ops.segment_sum(h[src], dst, num_segments=N)
    cnt = jax.ops.segment_sum(jnp.ones(src.shape[0], jnp.float32), dst, num_segments=N)
    mean = ssum / jnp.clip(cnt, 1.0)[:, None]
    sage_out = mean @ p['W_sage_nbr'] + h @ p['W_sage_root'] + p['b_sage']
    hx = jax.nn.elu(gat_out) + jax.nn.elu(gcn_out) + jax.nn.elu(sage_out)
    for blk in p['blocks']:
        t = _bn(hx @ blk['W1'] + blk['b1'], blk['g1'], blk['be1'])
        t = jax.nn.relu(t)
        t = _bn(t @ blk['W2'] + blk['b2'], blk['g2'], blk['be2'])
        hx = jax.nn.relu(t + hx)
    c = jax.nn.relu(hx @ p['W_c1'] + p['b_c1'])
    c = _bn(c, p['bn_c_g'], p['bn_c_b'])
    out = pl.pallas_call(
        _final_mm_kernel,
        out_shape=jax.ShapeDtypeStruct((N, 1), jnp.float32),
    )(c, p['W_c2'], p['b_c2'])
    return out.squeeze(-1)


# speedup vs baseline: 10.5861x; 10.5861x over previous
import functools

import jax
import jax.numpy as jnp
from jax import lax
from jax.experimental import pallas as pl
from jax.experimental.pallas import tpu as pltpu
from jax.experimental.pallas import tpu_sc as plsc

NN = 10000
EE = 160000
DIN = 1280
HH = 512
NHEAD = 4
HDIM = 128

RB = 1000
NRB = NN // RB
NC = 2
NS = 16
NW = NC * NS
ET1 = EE // NW
ET1P = 5120
NCH1 = ET1P // 16
ET2 = EE // NS
KB = 80
NB2 = ET2 // KB
NPAD = 10240
RPT = NPAD // NS
ZROWS = 128
CW = 64
RB3 = 400
NRB3 = NN // RB3
NCHK = HH // CW

_EPS = 1e-5
_PREC = lax.Precision.DEFAULT


def _bn_from_stats(x, stats_ref, g_ref, be_ref):
    mu = stats_ref[0:1] * (1.0 / NN)
    var = stats_ref[1:2] * (1.0 / NN) - mu * mu
    return g_ref[...] * (x - mu) * lax.rsqrt(var + _EPS) + be_ref[...]


def _acc_stats(y, i, acc_ref, stats_ref):
    @pl.when(i == 0)
    def _():
        acc_ref[...] = jnp.zeros_like(acc_ref)

    acc_ref[0:1] += jnp.sum(y, axis=0, keepdims=True)
    acc_ref[1:2] += jnp.sum(y * y, axis=0, keepdims=True)

    @pl.when(i == NRB - 1)
    def _():
        stats_ref[...] = acc_ref[...]


def _mmstats_body(x_ref, w_ref, b_ref, y_ref, stats_ref, acc_ref):
    i = pl.program_id(0)
    y = jnp.dot(x_ref[...], w_ref[...], preferred_element_type=jnp.float32,
                 precision=_PREC)
    y = y + b_ref[...]
    y_ref[...] = y
    _acc_stats(y, i, acc_ref, stats_ref)


def _mmstats(x, w, b):
    k1, k2 = w.shape
    return pl.pallas_call(
        _mmstats_body,
        grid=(NRB,),
        in_specs=[
            pl.BlockSpec((RB, k1), lambda i: (i, 0)),
            pl.BlockSpec((k1, k2), lambda i: (0, 0)),
            pl.BlockSpec((1, k2), lambda i: (0, 0)),
        ],
        out_specs=[
            pl.BlockSpec((RB, k2), lambda i: (i, 0)),
            pl.BlockSpec((8, k2), lambda i: (0, 0)),
        ],
        out_shape=[
            jax.ShapeDtypeStruct((NN, k2), jnp.float32),
            jax.ShapeDtypeStruct((8, k2), jnp.float32),
        ],
        scratch_shapes=[pltpu.VMEM((8, k2), jnp.float32)],
    )(x, w, b.reshape(1, k2))


def _bnrelu_mmstats_body(y_ref, st_ref, g_ref, be_ref, w_ref, b_ref,
                         y2_ref, st2_ref, acc_ref):
    i = pl.program_id(0)
    t = jnp.maximum(_bn_from_stats(y_ref[...], st_ref, g_ref, be_ref), 0.0)
    y2 = jnp.dot(t, w_ref[...], preferred_element_type=jnp.float32,
                 precision=_PREC) + b_ref[...]
    y2_ref[...] = y2
    _acc_stats(y2, i, acc_ref, st2_ref)


def _bnrelu_mmstats(y, stats, g, be, w, b):
    k1, k2 = w.shape
    return pl.pallas_call(
        _bnrelu_mmstats_body,
        grid=(NRB,),
        in_specs=[
            pl.BlockSpec((RB, k1), lambda i: (i, 0)),
            pl.BlockSpec((8, k1), lambda i: (0, 0)),
            pl.BlockSpec((1, k1), lambda i: (0, 0)),
            pl.BlockSpec((1, k1), lambda i: (0, 0)),
            pl.BlockSpec((k1, k2), lambda i: (0, 0)),
            pl.BlockSpec((1, k2), lambda i: (0, 0)),
        ],
        out_specs=[
            pl.BlockSpec((RB, k2), lambda i: (i, 0)),
            pl.BlockSpec((8, k2), lambda i: (0, 0)),
        ],
        out_shape=[
            jax.ShapeDtypeStruct((NN, k2), jnp.float32),
            jax.ShapeDtypeStruct((8, k2), jnp.float32),
        ],
        scratch_shapes=[pltpu.VMEM((8, k2), jnp.float32)],
    )(y, stats, g.reshape(1, k1), be.reshape(1, k1), w, b.reshape(1, k2))


def _bnresrelu_mmstats_body(relu_out, y_ref, st_ref, g_ref, be_ref, res_ref,
                            w_ref, b_ref, hnew_ref, y2_ref, st2_ref, acc_ref):
    i = pl.program_id(0)
    hnew = jnp.maximum(
        _bn_from_stats(y_ref[...], st_ref, g_ref, be_ref) + res_ref[...], 0.0)
    hnew_ref[...] = hnew
    y2 = jnp.dot(hnew, w_ref[...], preferred_element_type=jnp.float32,
                 precision=_PREC)
    y2 = y2 + b_ref[...]
    if relu_out:
        y2 = jnp.maximum(y2, 0.0)
    y2_ref[...] = y2
    _acc_stats(y2, i, acc_ref, st2_ref)


def _bnresrelu_mmstats(y, stats, g, be, res, w, b, relu_out):
    k1, k2 = w.shape
    return pl.pallas_call(
        functools.partial(_bnresrelu_mmstats_body, relu_out),
        grid=(NRB,),
        in_specs=[
            pl.BlockSpec((RB, k1), lambda i: (i, 0)),
            pl.BlockSpec((8, k1), lambda i: (0, 0)),
            pl.BlockSpec((1, k1), lambda i: (0, 0)),
            pl.BlockSpec((1, k1), lambda i: (0, 0)),
            pl.BlockSpec((RB, k1), lambda i: (i, 0)),
            pl.BlockSpec((k1, k2), lambda i: (0, 0)),
            pl.BlockSpec((1, k2), lambda i: (0, 0)),
        ],
        out_specs=[
            pl.BlockSpec((RB, k1), lambda i: (i, 0)),
            pl.BlockSpec((RB, k2), lambda i: (i, 0)),
            pl.BlockSpec((8, k2), lambda i: (0, 0)),
        ],
        out_shape=[
            jax.ShapeDtypeStruct((NN, k1), jnp.float32),
            jax.ShapeDtypeStruct((NN, k2), jnp.float32),
            jax.ShapeDtypeStruct((8, k2), jnp.float32),
        ],
        scratch_shapes=[pltpu.VMEM((8, k2), jnp.float32)],
    )(y, stats, g.reshape(1, k1), be.reshape(1, k1), res, w, b.reshape(1, k2))


def _tc1b_body(y_ref, st_ref, g_ref, be_ref, wg_ref, asrc_ref, adst_ref,
               h_ref, hc_ref, hgc_ref, ast_ref, adt_ref):
    h = jnp.maximum(_bn_from_stats(y_ref[...], st_ref, g_ref, be_ref), 0.0)
    h_ref[...] = h
    hg = jnp.dot(h, wg_ref[...], preferred_element_type=jnp.float32,
                 precision=_PREC)
    for c in range(NHEAD):
        hgc = hg[:, c * HDIM:(c + 1) * HDIM]
        ast_ref[:, c:c + 1] = jnp.sum(
            hgc * asrc_ref[c:c + 1, :], axis=1, keepdims=True)
        adt_ref[:, c:c + 1] = jnp.sum(
            hgc * adst_ref[c:c + 1, :], axis=1, keepdims=True)
    for k in range(NCHK):
        hc_ref[k] = h[:, k * CW:(k + 1) * CW]
        hgc_ref[k] = hg[:, k * CW:(k + 1) * CW]


def _tc1b(y, stats, g, be, w_gat, att_src, att_dst):
    return pl.pallas_call(
        _tc1b_body,
        grid=(NRB,),
        in_specs=[
            pl.BlockSpec((RB, HH), lambda i: (i, 0)),
            pl.BlockSpec((8, HH), lambda i: (0, 0)),
            pl.BlockSpec((1, HH), lambda i: (0, 0)),
            pl.BlockSpec((1, HH), lambda i: (0, 0)),
            pl.BlockSpec((HH, HH), lambda i: (0, 0)),
            pl.BlockSpec((NHEAD, HDIM), lambda i: (0, 0)),
            pl.BlockSpec((NHEAD, HDIM), lambda i: (0, 0)),
        ],
        out_specs=[
            pl.BlockSpec((RB, HH), lambda i: (i, 0)),
            pl.BlockSpec((NCHK, RB, CW), lambda i: (0, i, 0)),
            pl.BlockSpec((NCHK, RB, CW), lambda i: (0, i, 0)),
            pl.BlockSpec((RB, NHEAD), lambda i: (i, 0)),
            pl.BlockSpec((RB, NHEAD), lambda i: (i, 0)),
        ],
        out_shape=[
            jax.ShapeDtypeStruct((NN, HH), jnp.float32),
            jax.ShapeDtypeStruct((NCHK, NN, CW), jnp.float32),
            jax.ShapeDtypeStruct((NCHK, NN, CW), jnp.float32),
            jax.ShapeDtypeStruct((NN, NHEAD), jnp.float32),
            jax.ShapeDtypeStruct((NN, NHEAD), jnp.float32),
        ],
        scratch_shapes=[],
    )(y, stats, g.reshape(1, HH), be.reshape(1, HH), w_gat, att_src, att_dst)


def _tc2a_body(denomp_ref, cntp_ref, ast_ref, adt_ref,
               dt_ref, exs_ref, dinv_ref, cnt_ref):
    cnt = cntp_ref[0:1]
    for i in range(1, NW):
        cnt = cnt + cntp_ref[i:i + 1]
    cnt_ref[...] = cnt
    dinv = lax.rsqrt(cnt + 1.0)
    dinv_ref[...] = dinv
    de = denomp_ref[0]
    for i in range(1, NW):
        de = de + denomp_ref[i]
    es = ast_ref[...] + adt_ref[...]
    es = jnp.where(es >= 0.0, es, es * 0.2)
    es = jnp.clip(es, -60.0, 60.0)
    es = jnp.exp(es)
    exs_ref[...] = es
    dt_ref[...] = de + es


def _tc2a(denomp, cntp, ast_t, adt_t):
    return pl.pallas_call(
        _tc2a_body,
        out_shape=[
            jax.ShapeDtypeStruct((NHEAD, NN), jnp.float32),
            jax.ShapeDtypeStruct((NHEAD, NN), jnp.float32),
            jax.ShapeDtypeStruct((1, NN), jnp.float32),
            jax.ShapeDtypeStruct((1, NN), jnp.float32),
        ],
    )(denomp, cntp, ast_t, adt_t)


def _tc2b_body(h_ref, dinv_ref, w_ref, hcn_ref):
    hc = jnp.dot(h_ref[...], w_ref[...], preferred_element_type=jnp.float32,
                 precision=_PREC)
    hcn = hc * dinv_ref[...]
    for k in range(NCHK):
        hcn_ref[k] = hcn[:, k * CW:(k + 1) * CW]


def _tc2b(h, dinv_n1, w_gcn):
    return pl.pallas_call(
        _tc2b_body,
        grid=(NRB,),
        in_specs=[
            pl.BlockSpec((RB, HH), lambda i: (i, 0)),
            pl.BlockSpec((RB, 1), lambda i: (i, 0)),
            pl.BlockSpec((HH, HH), lambda i: (0, 0)),
        ],
        out_specs=pl.BlockSpec((NCHK, RB, CW), lambda i: (0, i, 0)),
        out_shape=jax.ShapeDtypeStruct((NCHK, NN, CW), jnp.float32),
    )(h, dinv_n1, w_gcn)


def _tc3a_body(g_ref, h_ref, hgc_ref, hcn_ref, dt_ref, exs_ref, dinv_ref,
               cnt_ref, wsn_ref, wsr_ref, bgat_ref, bgcn_ref, bsage_ref,
               hx_ref):
    h = h_ref[...]
    sage_rows = jnp.concatenate([g_ref[k] for k in range(NCHK)], axis=1)
    gcn_rows = jnp.concatenate([g_ref[8 + k] for k in range(NCHK)], axis=1)
    cnt = jnp.maximum(cnt_ref[...], 1.0)
    mean = sage_rows / cnt
    sage_out = (jnp.dot(mean, wsn_ref[...], preferred_element_type=jnp.float32,
                 precision=_PREC)
                + jnp.dot(h, wsr_ref[...], preferred_element_type=jnp.float32,
                 precision=_PREC)
                + bsage_ref[...])
    dinv = dinv_ref[...]
    hcn_rows = jnp.concatenate([hcn_ref[k] for k in range(NCHK)], axis=1)
    gcn_out = dinv * (gcn_rows + hcn_rows) + bgcn_ref[...]
    gat_parts = []
    for c in range(NHEAD):
        gagg = jnp.concatenate([g_ref[16 + 2 * c], g_ref[17 + 2 * c]], axis=1)
        hgcc = jnp.concatenate([hgc_ref[2 * c], hgc_ref[2 * c + 1]], axis=1)
        num = gagg + exs_ref[:, c:c + 1] * hgcc
        gat_parts.append(num / (dt_ref[:, c:c + 1] + 1e-16))
    gat_out = jnp.concatenate(gat_parts, axis=1) + bgat_ref[...]

    def elu(v):
        return jnp.where(v > 0.0, v, jnp.exp(jnp.minimum(v, 0.0)) - 1.0)

    hx_ref[...] = elu(gat_out) + elu(gcn_out) + elu(sage_out)


def _tc3a(g_agg, h, hgc_all, hcn_all, dt_n4, exs_n4, dinv_n1, cnt_n1,
          w_sn, w_sr, b_gat, b_gcn, b_sage):
    return pl.pallas_call(
        _tc3a_body,
        grid=(NRB3,),
        in_specs=[
            pl.BlockSpec((24, RB3, CW), lambda i: (0, i, 0)),
            pl.BlockSpec((RB3, HH), lambda i: (i, 0)),
            pl.BlockSpec((NCHK, RB3, CW), lambda i: (0, i, 0)),
            pl.BlockSpec((NCHK, RB3, CW), lambda i: (0, i, 0)),
            pl.BlockSpec((RB3, NHEAD), lambda i: (i, 0)),
            pl.BlockSpec((RB3, NHEAD), lambda i: (i, 0)),
            pl.BlockSpec((RB3, 1), lambda i: (i, 0)),
            pl.BlockSpec((RB3, 1), lambda i: (i, 0)),
            pl.BlockSpec((HH, HH), lambda i: (0, 0)),
            pl.BlockSpec((HH, HH), lambda i: (0, 0)),
            pl.BlockSpec((1, HH), lambda i: (0, 0)),
            pl.BlockSpec((1, HH), lambda i: (0, 0)),
            pl.BlockSpec((1, HH), lambda i: (0, 0)),
        ],
        out_specs=pl.BlockSpec((RB3, HH), lambda i: (i, 0)),
        out_shape=jax.ShapeDtypeStruct((NN, HH), jnp.float32),
    )(g_agg, h, hgc_all, hcn_all, dt_n4, exs_n4, dinv_n1, cnt_n1,
      w_sn, w_sr, b_gat.reshape(1, HH), b_gcn.reshape(1, HH),
      b_sage.reshape(1, HH))


def _final_body(c_ref, st_ref, g_ref, be_ref, wrow_ref, b_ref, o_ref):
    cb = _bn_from_stats(c_ref[...], st_ref, g_ref, be_ref)
    o_ref[...] = jnp.sum(cb * wrow_ref[...], axis=1, keepdims=True) + b_ref[...]


def _final(c, stats, g, be, w_row, b):
    return pl.pallas_call(
        _final_body,
        grid=(NRB,),
        in_specs=[
            pl.BlockSpec((RB, 256), lambda i: (i, 0)),
            pl.BlockSpec((8, 256), lambda i: (0, 0)),
            pl.BlockSpec((1, 256), lambda i: (0, 0)),
            pl.BlockSpec((1, 256), lambda i: (0, 0)),
            pl.BlockSpec((1, 256), lambda i: (0, 0)),
            pl.BlockSpec((1, 1), lambda i: (0, 0)),
        ],
        out_specs=pl.BlockSpec((RB, 1), lambda i: (i, 0)),
        out_shape=jax.ShapeDtypeStruct((NN, 1), jnp.float32),
    )(c, stats, g.reshape(1, 256), be.reshape(1, 256), w_row, b.reshape(1, 1))


def _k1_body(src_hbm, dst_hbm, ast_hbm, adt_hbm,
             ex_hbm, denomp_hbm, cntp_hbm,
             src_v, dst_v, tab_s, tab_d, acc_v, cnt_v, ex_v):
    cid = lax.axis_index("c")
    sid = lax.axis_index("s")
    wid = sid * NC + cid

    z16f = jnp.zeros((16,), jnp.float32)
    ones16 = jnp.ones((16,), jnp.float32)
    iota16 = lax.iota(jnp.int32, 16)

    pltpu.sync_copy(src_hbm.at[wid], src_v)
    pltpu.sync_copy(dst_hbm.at[wid], dst_v)

    def zero_tab(ref):
        def zb(k, _):
            ref[pl.ds(k * 16, 16)] = z16f
            return 0
        lax.fori_loop(0, NN // 16, zb, 0)

    zero_tab(cnt_v)

    for hd in range(NHEAD):
        pltpu.sync_copy(ast_hbm.at[hd], tab_s)
        pltpu.sync_copy(adt_hbm.at[hd], tab_d)
        zero_tab(acc_v)

        def chunk(j, _):
            s16 = src_v[pl.ds(j * 16, 16)]
            d16 = dst_v[pl.ds(j * 16, 16)]
            mask = (iota16 + j * 16) < ET1
            av = plsc.load_gather(tab_s, [s16])
            bv = plsc.load_gather(tab_d, [d16])
            e = av + bv
            e = jnp.where(e >= 0.0, e, e * 0.2)
            e = jnp.clip(e, -60.0, 60.0)
            exv = jnp.exp(e)
            for l in range(16):
                lm = jnp.logical_and(mask, iota16 == l)
                plsc.addupdate_scatter(acc_v, [d16], exv, mask=lm)
                if hd == 0:
                    plsc.addupdate_scatter(cnt_v, [d16], ones16, mask=lm)
            ex_v[pl.ds(j * 16, 16)] = jnp.where(mask, exv, 0.0)
            return 0

        lax.fori_loop(0, NCH1, chunk, 0)

        pltpu.sync_copy(ex_v, ex_hbm.at[hd].at[wid])
        pltpu.sync_copy(acc_v, denomp_hbm.at[wid].at[hd])

    pltpu.sync_copy(cnt_v, cntp_hbm.at[wid])


def _k1(src2, dst2, ast_t, adt_t):
    mesh = plsc.VectorSubcoreMesh(core_axis_name="c", subcore_axis_name="s")
    f = pl.kernel(
        _k1_body,
        compiler_params=pltpu.CompilerParams(
            use_tc_tiling_on_sc=False, needs_layout_passes=False),
        out_type=[
            jax.ShapeDtypeStruct((NHEAD, NW, ET1P), jnp.float32),
            jax.ShapeDtypeStruct((NW, NHEAD, NN), jnp.float32),
            jax.ShapeDtypeStruct((NW, NN), jnp.float32),
        ],
        mesh=mesh,
        scratch_types=[
            pltpu.VMEM((ET1P,), jnp.int32),
            pltpu.VMEM((ET1P,), jnp.int32),
            pltpu.VMEM((NN,), jnp.float32),
            pltpu.VMEM((NN,), jnp.float32),
            pltpu.VMEM((NN,), jnp.float32),
            pltpu.VMEM((NN,), jnp.float32),
            pltpu.VMEM((ET1P,), jnp.float32),
        ],
    )
    return f(src2, dst2, ast_t, adt_t)


def _k2_body(hc_hbm, hpc_hbm, hgc_hbm, ex4_hbm, src3_hbm, dst3_hbm,
             g_hbm,
             src2_v, dst2_v, exw_v, rows_v, zbuf_v, acc_sh, sem):
    cid = lax.axis_index("c")
    sid = lax.axis_index("s")

    z16f = jnp.zeros((16,), jnp.float32)
    iota16 = lax.iota(jnp.int32, 16)

    def zb(k, _):
        zbuf_v[k // 8, pl.ds((k % 8) * 16, 16)] = z16f
        return 0
    lax.fori_loop(0, ZROWS * 8, zb, 0)

    pltpu.sync_copy(src3_hbm.at[sid], src2_v)
    pltpu.sync_copy(dst3_hbm.at[sid], dst2_v)

    def run_pass(table_hbm, chunk, slot, ex_head):
        for k in range(RPT // ZROWS):
            pltpu.sync_copy(zbuf_v,
                            acc_sh.at[pl.ds(sid * RPT + k * ZROWS, ZROWS)])
        plsc.subcore_barrier()
        if ex_head is not None:
            pltpu.sync_copy(ex4_hbm.at[ex_head].at[sid], exw_v)

        def batch(j, _):
            pltpu.async_copy(table_hbm.at[chunk].at[src2_v.at[j]],
                             rows_v, sem).wait()
            if ex_head is not None:
                for g in range(KB // 16):
                    w16g = exw_v[pl.ds(j * KB + g * 16, 16)]
                    for l in range(16):
                        w16 = jnp.full((16,), 1.0, jnp.float32) * w16g[l]
                        r = g * 16 + l
                        for f in range(CW // 16):
                            v = rows_v[r, pl.ds(f * 16, 16)]
                            rows_v[r, pl.ds(f * 16, 16)] = v * w16
            pltpu.sync_copy(rows_v, acc_sh.at[dst2_v.at[j]], add=True)
            return 0

        lax.fori_loop(0, NB2, batch, 0)
        plsc.subcore_barrier()
        pltpu.sync_copy(acc_sh.at[pl.ds(sid * RPT, RPT)],
                        g_hbm.at[slot].at[pl.ds(sid * RPT, RPT)])

    passes_c0 = ([(hc_hbm, k, k, None) for k in range(NCHK)]
                 + [(hgc_hbm, k, 16 + k, k // 2) for k in range(4)])
    passes_c1 = ([(hpc_hbm, k, 8 + k, None) for k in range(NCHK)]
                 + [(hgc_hbm, k, 16 + k, k // 2) for k in range(4, 8)])

    @pl.when(cid == 0)
    def _():
        for args in passes_c0:
            run_pass(*args)

    @pl.when(cid == 1)
    def _():
        for args in passes_c1:
            run_pass(*args)


def _k2(hc_all, hpc_all, hgc_all, ex4, src3, dst3):
    mesh = plsc.VectorSubcoreMesh(core_axis_name="c", subcore_axis_name="s")
    f = pl.kernel(
        _k2_body,
        compiler_params=pltpu.CompilerParams(
            use_tc_tiling_on_sc=False, needs_layout_passes=False),
        out_type=[
            jax.ShapeDtypeStruct((24, NPAD, CW), jnp.float32),
        ],
        mesh=mesh,
        scratch_types=[
            pltpu.VMEM((NB2, KB), jnp.int32),
            pltpu.VMEM((NB2, KB), jnp.int32),
            pltpu.VMEM((ET2,), jnp.float32),
            pltpu.VMEM((KB, CW), jnp.float32),
            pltpu.VMEM((ZROWS, CW), jnp.float32),
            pltpu.VMEM_SHARED((NPAD, CW), jnp.float32),
            pltpu.SemaphoreType.DMA,
        ],
    )
    return f(hc_all, hpc_all, hgc_all, ex4, src3, dst3)[0]


def kernel(x, edge_index, params):
    p = params
    src = edge_index[0]
    dst = edge_index[1]

    y_in, st_in = _mmstats(x, p['W_in'], p['b_in'])
    h, hc_all, hgc_all, ast, adt = _tc1b(
        y_in, st_in, p['bn_in_g'], p['bn_in_b'], p['W_gat'],
        p['att_src'], p['att_dst'])

    ast_t = ast.T
    adt_t = adt.T

    epad = ((0, 0), (0, ET1P - ET1))
    srcp = jnp.pad(src.reshape(NW, ET1), epad)
    dstp = jnp.pad(dst.reshape(NW, ET1), epad)
    ex, denomp, cntp = _k1(srcp, dstp, ast_t, adt_t)

    dt_t, exs_t, dinv_t, cnt_t = _tc2a(denomp, cntp, ast_t, adt_t)
    dinv_n1 = dinv_t.T
    cnt_n1 = cnt_t.T
    dt_n4 = dt_t.T
    exs_n4 = exs_t.T

    hpc_all = _tc2b(h, dinv_n1, p['W_gcn'])

    ex4 = ex[:, :, :ET1].reshape(NHEAD, NS, ET2)
    src3 = src.reshape(NS, NB2, KB)
    dst3 = dst.reshape(NS, NB2, KB)
    g_agg = _k2(hc_all, hpc_all, hgc_all, ex4, src3, dst3)

    hx0 = _tc3a(g_agg, h, hgc_all, hpc_all, dt_n4, exs_n4, dinv_n1, cnt_n1,
                p['W_sage_nbr'], p['W_sage_root'],
                p['b_gat'], p['b_gcn'], p['b_sage'])

    b0, b1 = p['blocks']
    y1, s1 = _mmstats(hx0, b0['W1'], b0['b1'])
    y2, s2 = _bnrelu_mmstats(y1, s1, b0['g1'], b0['be1'], b0['W2'], b0['b2'])
    hx1, y3, s3 = _bnresrelu_mmstats(y2, s2, b0['g2'], b0['be2'], hx0,
                                     b1['W1'], b1['b1'], relu_out=False)
    y4, s4 = _bnrelu_mmstats(y3, s3, b1['g1'], b1['be1'], b1['W2'], b1['b2'])
    _, c5, s5 = _bnresrelu_mmstats(y4, s4, b1['g2'], b1['be2'], hx1,
                                   p['W_c1'], p['b_c1'], relu_out=True)
    out = _final(c5, s5, p['bn_c_g'], p['bn_c_b'],
                 p['W_c2'].reshape(1, 256), p['b_c2'])
    return out.reshape(NN)
